# baseline (speedup 1.0000x reference)
import functools

import jax
import jax.numpy as jnp
from jax import lax
from jax.experimental import pallas as pl
from jax.experimental.pallas import tpu as pltpu
from jax.experimental.pallas import tpu_sc as plsc

NC = 2
NS = 16
NW = NC * NS
L = 16
CH = 2048


def _make_sc_kernel(n, epad, hpad, zpad):
    e_chunks = epad // (NW * CH)
    h_chunks = hpad // (NW * CH)

    mesh = plsc.VectorSubcoreMesh(
        core_axis_name="c", subcore_axis_name="s", num_cores=NC,
        num_subcores=NS)

    @functools.partial(
        pl.kernel,
        out_type=[jax.ShapeDtypeStruct((NC, zpad), jnp.float32),
                  jax.ShapeDtypeStruct((NC, zpad), jnp.float32)],
        mesh=mesh,
        scratch_types=[
            pltpu.VMEM((CH,), jnp.int32),
            pltpu.VMEM((CH,), jnp.int32),
            pltpu.VMEM((CH,), jnp.int32),
            pltpu.VMEM((CH,), jnp.float32),
            pltpu.VMEM((CH,), jnp.float32),
            pltpu.VMEM((CH,), jnp.float32),
            pltpu.VMEM((CH,), jnp.float32),
            pltpu.VMEM((zpad,), jnp.float32),
            pltpu.MemorySpace.VMEM_SHARED((zpad,), jnp.float32),
            pltpu.MemorySpace.VMEM_SHARED((zpad,), jnp.float32),
        ],
        compiler_params=pltpu.CompilerParams(needs_layout_passes=False),
    )
    def sc_kernel(x_hbm, src_hbm, dst_hbm, h1_hbm, h2_hbm, h3_hbm, z_hbm,
                  out_hbm, deg_hbm, ia, ib, ic, ga, gb, gc, ones_v, x_sh,
                  acc, dega):
        c = lax.axis_index("c")
        s = lax.axis_index("s")
        wid = c * NS + s

        @pl.when(s == 0)
        def _zero():
            pltpu.sync_copy(z_hbm, acc)

        @pl.when(s == 1)
        def _zerod():
            pltpu.sync_copy(z_hbm, dega)

        pltpu.sync_copy(x_hbm, x_sh)

        def fill(j, carry):
            ones_v[pl.ds(j * L, L)] = jnp.full((L,), 1.0, jnp.float32)
            return carry
        lax.fori_loop(0, CH // L, fill, 0)

        plsc.subcore_barrier()

        def edge_chunk(k, carry):
            s0 = (k * NW + wid) * CH
            pltpu.sync_copy(src_hbm.at[pl.ds(s0, CH)], ia)
            pltpu.sync_copy(dst_hbm.at[pl.ds(s0, CH)], ib)

            def vec(j, carry2):
                sl = pl.ds(j * L, L)
                xs = plsc.load_gather(x_sh, [ia[sl]])
                ga[sl] = 0.1 * xs
                return carry2
            lax.fori_loop(0, CH // L, vec, carry)
            pltpu.sync_copy(ga, acc.at[ib], add=True)
            pltpu.sync_copy(ones_v, dega.at[ib], add=True)
            return carry
        lax.fori_loop(0, e_chunks, edge_chunk, 0)

        def hyper_chunk(k, carry):
            s0 = (k * NW + wid) * CH
            pltpu.sync_copy(h1_hbm.at[pl.ds(s0, CH)], ia)
            pltpu.sync_copy(h2_hbm.at[pl.ds(s0, CH)], ib)
            pltpu.sync_copy(h3_hbm.at[pl.ds(s0, CH)], ic)

            def vec(j, carry2):
                sl = pl.ds(j * L, L)
                a = plsc.load_gather(x_sh, [ia[sl]])
                b = plsc.load_gather(x_sh, [ib[sl]])
                cc = plsc.load_gather(x_sh, [ic[sl]])
                p = b * cc
                ga[sl] = 0.9 * (p - a * a)
                gb[sl] = 0.9 * (p - b * b)
                gc[sl] = 0.9 * (p - cc * cc)
                return carry2
            lax.fori_loop(0, CH // L, vec, carry)
            pltpu.sync_copy(ga, acc.at[ia], add=True)
            pltpu.sync_copy(gb, acc.at[ib], add=True)
            pltpu.sync_copy(gc, acc.at[ic], add=True)
            return carry
        lax.fori_loop(0, h_chunks, hyper_chunk, 0)

        plsc.subcore_barrier()

        @pl.when(s == 0)
        def _out():
            pltpu.sync_copy(acc, out_hbm.at[c])

        @pl.when(s == 1)
        def _outd():
            pltpu.sync_copy(dega, deg_hbm.at[c])

    return sc_kernel


def _final_body(parts_ref, degs_ref, x_ref, o_ref):
    deg = degs_ref[0, :] + degs_ref[1, :]
    o_ref[...] = parts_ref[0, :] + parts_ref[1, :] - 0.1 * deg * x_ref[...]


def kernel(t, x, edge_index, hyperedges):
    del t
    n = x.shape[0]
    e = edge_index.shape[1]
    h = hyperedges.shape[0]
    blk = NW * CH
    zpad = -(-n // (NS * 128)) * (NS * 128)
    epad = -(-e // blk) * blk
    hpad = -(-h // blk) * blk

    ei = edge_index.astype(jnp.int32)
    he = hyperedges.astype(jnp.int32)

    src = jnp.pad(ei[0], (0, epad - e))
    dst = jnp.pad(ei[1], (0, epad - e), constant_values=n)
    h1 = jnp.pad(he[:, 0], (0, hpad - h))
    h2 = jnp.pad(he[:, 1], (0, hpad - h))
    h3 = jnp.pad(he[:, 2], (0, hpad - h))

    z = jnp.zeros((zpad,), jnp.float32)
    xp = jnp.pad(x, (0, zpad - n))

    parts, degs = _make_sc_kernel(n, epad, hpad, zpad)(
        xp, src, dst, h1, h2, h3, z)

    summed = pl.pallas_call(
        _final_body,
        out_shape=jax.ShapeDtypeStruct((zpad,), jnp.float32),
    )(parts, degs, xp)
    return summed[:n]


# speedup vs baseline: 1.1343x; 1.1343x over previous
import functools

import jax
import jax.numpy as jnp
from jax import lax
from jax.experimental import pallas as pl
from jax.experimental.pallas import tpu as pltpu
from jax.experimental.pallas import tpu_sc as plsc

NC = 2
NS = 16
NW = NC * NS
L = 16
CH = 2048


def _make_sc_kernel(n, epad, hpad, zpad):
    e_chunks = epad // (NW * CH)
    h_chunks = hpad // (NW * CH)

    mesh = plsc.VectorSubcoreMesh(
        core_axis_name="c", subcore_axis_name="s", num_cores=NC,
        num_subcores=NS)

    @functools.partial(
        pl.kernel,
        out_type=jax.ShapeDtypeStruct((NC, zpad), jnp.float32),
        mesh=mesh,
        scratch_types=[
            pltpu.VMEM((CH,), jnp.int32),
            pltpu.VMEM((CH,), jnp.int32),
            pltpu.VMEM((CH,), jnp.int32),
            pltpu.VMEM((CH,), jnp.float32),
            pltpu.VMEM((CH,), jnp.float32),
            pltpu.VMEM((CH,), jnp.float32),
            pltpu.VMEM((zpad,), jnp.float32),
            pltpu.MemorySpace.VMEM_SHARED((zpad,), jnp.float32),
        ],
        compiler_params=pltpu.CompilerParams(needs_layout_passes=False),
    )
    def sc_kernel(x_hbm, src_hbm, dst_hbm, h1_hbm, h2_hbm, h3_hbm, z_hbm,
                  out_hbm, ia, ib, ic, ga, gb, gc, x_v, acc):
        c = lax.axis_index("c")
        s = lax.axis_index("s")
        wid = c * NS + s

        @pl.when(s == 0)
        def _zero():
            pltpu.sync_copy(z_hbm, acc)

        pltpu.sync_copy(x_hbm, x_v)

        plsc.subcore_barrier()

        def edge_chunk(k, carry):
            s0 = (k * NW + wid) * CH
            pltpu.sync_copy(src_hbm.at[pl.ds(s0, CH)], ia)
            pltpu.sync_copy(dst_hbm.at[pl.ds(s0, CH)], ib)

            def vec(j, carry2):
                sl = pl.ds(j * L, L)
                xs = plsc.load_gather(x_v, [ia[sl]])
                xd = plsc.load_gather(x_v, [ib[sl]])
                ga[sl] = -0.1 * (xd - xs)
                return carry2
            lax.fori_loop(0, CH // L, vec, carry)
            pltpu.sync_copy(ga, acc.at[ib], add=True)
            return carry
        lax.fori_loop(0, e_chunks, edge_chunk, 0)

        def hyper_chunk(k, carry):
            s0 = (k * NW + wid) * CH
            pltpu.sync_copy(h1_hbm.at[pl.ds(s0, CH)], ia)
            pltpu.sync_copy(h2_hbm.at[pl.ds(s0, CH)], ib)
            pltpu.sync_copy(h3_hbm.at[pl.ds(s0, CH)], ic)

            def vec(j, carry2):
                sl = pl.ds(j * L, L)
                a = plsc.load_gather(x_v, [ia[sl]])
                b = plsc.load_gather(x_v, [ib[sl]])
                cc = plsc.load_gather(x_v, [ic[sl]])
                p = b * cc
                ga[sl] = 0.9 * (p - a * a)
                gb[sl] = 0.9 * (p - b * b)
                gc[sl] = 0.9 * (p - cc * cc)
                return carry2
            lax.fori_loop(0, CH // L, vec, carry)
            pltpu.sync_copy(ga, acc.at[ia], add=True)
            pltpu.sync_copy(gb, acc.at[ib], add=True)
            pltpu.sync_copy(gc, acc.at[ic], add=True)
            return carry
        lax.fori_loop(0, h_chunks, hyper_chunk, 0)

        plsc.subcore_barrier()

        @pl.when(s == 0)
        def _out():
            pltpu.sync_copy(acc, out_hbm.at[c])

    return sc_kernel


def _sum2_body(parts_ref, o_ref):
    o_ref[...] = parts_ref[0, :] + parts_ref[1, :]


def kernel(t, x, edge_index, hyperedges):
    del t
    n = x.shape[0]
    e = edge_index.shape[1]
    h = hyperedges.shape[0]
    blk = NW * CH
    zpad = -(-n // (NS * 128)) * (NS * 128)
    epad = -(-e // blk) * blk
    hpad = -(-h // blk) * blk

    ei = edge_index.astype(jnp.int32)
    he = hyperedges.astype(jnp.int32)

    src = jnp.pad(ei[0], (0, epad - e))
    dst = jnp.pad(ei[1], (0, epad - e), constant_values=n)
    h1 = jnp.pad(he[:, 0], (0, hpad - h))
    h2 = jnp.pad(he[:, 1], (0, hpad - h))
    h3 = jnp.pad(he[:, 2], (0, hpad - h))

    z = jnp.zeros((zpad,), jnp.float32)
    xp = jnp.pad(x, (0, zpad - n))

    parts = _make_sc_kernel(n, epad, hpad, zpad)(xp, src, dst, h1, h2, h3, z)

    summed = pl.pallas_call(
        _sum2_body,
        out_shape=jax.ShapeDtypeStruct((zpad,), jnp.float32),
    )(parts)
    return summed[:n]
